# baseline (speedup 1.0000x reference)
import jax
import jax.numpy as jnp
from jax.experimental import pallas as pl

B, N, C = 64, 1024, 192


def _mask_body(br_ref, tb_ref, w_ref, bias_ref, mask_ref):
    logits = jax.lax.dot_general(
        br_ref[...], w_ref[...],
        dimension_numbers=(((1,), (1,)), ((), ())),
        preferred_element_type=jnp.float32) + bias_ref[...]

    x = logits + 0.0
    u = jax.lax.bitcast_convert_type(x, jnp.uint32)
    key = u ^ jnp.where((u >> 31) > 0,
                        jnp.uint32(0xFFFFFFFF), jnp.uint32(0x80000000))

    K = jnp.clip((tb_ref[...] * float(N)).astype(jnp.int32), 1, N)

    t = jnp.zeros((B, 1), jnp.uint32)
    for bit in range(31, -1, -1):
        cand = t | jnp.uint32(1 << bit)
        cnt = jnp.sum((key >= cand).astype(jnp.int32), axis=1, keepdims=True)
        t = jnp.where(cnt >= K, cand, t)

    gt = key > t
    eq = key == t
    cnt_gt = jnp.sum(gt.astype(jnp.int32), axis=1, keepdims=True)
    need = (K - cnt_gt).astype(jnp.float32)

    ii = jax.lax.broadcasted_iota(jnp.int32, (N, N), 0)
    jj = jax.lax.broadcasted_iota(jnp.int32, (N, N), 1)
    ut = (ii < jj).astype(jnp.float32)
    cum_excl = jax.lax.dot_general(
        eq.astype(jnp.float32), ut,
        dimension_numbers=(((1,), (0,)), ((), ())),
        preferred_element_type=jnp.float32)

    keep = gt | (eq & (cum_excl < need))
    mask_ref[...] = keep.astype(jnp.float32)


def _mul_body(vis_ref, mask_ref, out_ref):
    patches = vis_ref[0, pl.ds(1, N), :]
    m = mask_ref[0, 0, :]
    out_ref[0] = patches * m[:, None]


def kernel(vision_output, token_budget, W, b):
    budget_repr = vision_output[:, -1, :]
    tb2 = token_budget.reshape(B, 1)
    b2 = b.reshape(1, N)

    keep_mask = pl.pallas_call(
        _mask_body,
        out_shape=jax.ShapeDtypeStruct((B, N), jnp.float32),
    )(budget_repr, tb2, W, b2)

    masked = pl.pallas_call(
        _mul_body,
        grid=(B,),
        in_specs=[
            pl.BlockSpec((1, N + 2, C), lambda i: (i, 0, 0)),
            pl.BlockSpec((1, 1, N), lambda i: (i, 0, 0)),
        ],
        out_specs=pl.BlockSpec((1, N, C), lambda i: (i, 0, 0)),
        out_shape=jax.ShapeDtypeStruct((B, N, C), jnp.float32),
    )(vision_output, keep_mask.reshape(B, 1, N))

    return masked, keep_mask


# speedup vs baseline: 2.4840x; 2.4840x over previous
import jax
import jax.numpy as jnp
from jax.experimental import pallas as pl

B, N, C = 64, 1024, 192


def _mask_body(br_ref, tb_ref, wt_ref, bias_ref, mask_ref):
    logits = jax.lax.dot_general(
        br_ref[...], wt_ref[...],
        dimension_numbers=(((1,), (0,)), ((), ())),
        preferred_element_type=jnp.float32) + bias_ref[...]

    x = logits + 0.0
    u = jax.lax.bitcast_convert_type(x, jnp.uint32)
    key = u ^ jnp.where((u >> 31) > 0,
                        jnp.uint32(0xFFFFFFFF), jnp.uint32(0x80000000))

    K = jnp.clip((tb_ref[...] * float(N)).astype(jnp.int32), 1, N)

    t = jnp.zeros((B, 1), jnp.uint32)
    for bit in range(31, -1, -1):
        cand = t | jnp.uint32(1 << bit)
        cnt = jnp.sum((key >= cand).astype(jnp.int32), axis=1, keepdims=True)
        t = jnp.where(cnt >= K, cand, t)

    gt = key > t
    eq = key == t
    cnt_gt = jnp.sum(gt.astype(jnp.int32), axis=1, keepdims=True)
    need = (K - cnt_gt).astype(jnp.float32)

    ii = jax.lax.broadcasted_iota(jnp.int32, (N, N), 0)
    jj = jax.lax.broadcasted_iota(jnp.int32, (N, N), 1)
    ut = (ii < jj).astype(jnp.float32)
    cum_excl = jax.lax.dot_general(
        eq.astype(jnp.float32), ut,
        dimension_numbers=(((1,), (0,)), ((), ())),
        preferred_element_type=jnp.float32)

    keep = gt | (eq & (cum_excl < need))
    mask_ref[...] = keep.astype(jnp.float32)


def _mul_body(vt_ref, mask_ref, out_ref):
    patches = vt_ref[0, :, pl.ds(1, N)]
    m = mask_ref[0, 0, :]
    out_ref[0] = patches * m[None, :]


def kernel(vision_output, token_budget, W, b):
    vt = jnp.transpose(vision_output, (0, 2, 1))
    budget_repr = vt[:, :, N + 1]
    tb2 = token_budget.reshape(B, 1)
    b2 = b.reshape(1, N)
    wt = W.T

    keep_mask = pl.pallas_call(
        _mask_body,
        out_shape=jax.ShapeDtypeStruct((B, N), jnp.float32),
    )(budget_repr, tb2, wt, b2)

    masked_t = pl.pallas_call(
        _mul_body,
        grid=(B,),
        in_specs=[
            pl.BlockSpec((1, C, N + 2), lambda i: (i, 0, 0)),
            pl.BlockSpec((1, 1, N), lambda i: (i, 0, 0)),
        ],
        out_specs=pl.BlockSpec((1, C, N), lambda i: (i, 0, 0)),
        out_shape=jax.ShapeDtypeStruct((B, C, N), jnp.float32),
    )(vt, keep_mask.reshape(B, 1, N))

    masked = jnp.transpose(masked_t, (0, 2, 1))
    return masked, keep_mask


# speedup vs baseline: 4.2850x; 1.7250x over previous
import jax
import jax.numpy as jnp
from jax.experimental import pallas as pl

B, N, C = 64, 1024, 192
LAST_TILE = (N + 2) // 128
LAST_OFF = (N + 1) % 128
B_BLK = 4


def _mask_body(vtail_ref, tb_ref, wt_ref, bias_ref, mask_ref):
    br = vtail_ref[:, :, LAST_OFF]
    logits = jax.lax.dot_general(
        br, wt_ref[...],
        dimension_numbers=(((1,), (0,)), ((), ())),
        preferred_element_type=jnp.float32) + bias_ref[...]

    ii = jax.lax.broadcasted_iota(jnp.int32, (B, B), 0)
    jj = jax.lax.broadcasted_iota(jnp.int32, (B, B), 1)
    tb_col = jnp.sum(jnp.where(ii == jj, jnp.broadcast_to(tb_ref[...], (B, B)),
                               0.0), axis=1, keepdims=True)
    K = jnp.clip((tb_col * float(N)).astype(jnp.int32), 1, N)

    x = logits + 0.0
    u = jax.lax.bitcast_convert_type(x, jnp.uint32)
    key = u ^ jnp.where((u >> 31) > 0,
                        jnp.uint32(0xFFFFFFFF), jnp.uint32(0x80000000))

    t = jnp.zeros((B, 1), jnp.uint32)
    for bit in range(31, -1, -1):
        cand = t | jnp.uint32(1 << bit)
        cnt = jnp.sum((key >= cand).astype(jnp.int32), axis=1, keepdims=True)
        t = jnp.where(cnt >= K, cand, t)

    gt = key > t
    eq = key == t
    cnt_gt = jnp.sum(gt.astype(jnp.int32), axis=1, keepdims=True)
    need = (K - cnt_gt).astype(jnp.float32)

    ni = jax.lax.broadcasted_iota(jnp.int32, (N, N), 0)
    nj = jax.lax.broadcasted_iota(jnp.int32, (N, N), 1)
    ut = (ni < nj).astype(jnp.float32)
    cum_excl = jax.lax.dot_general(
        eq.astype(jnp.float32), ut,
        dimension_numbers=(((1,), (0,)), ((), ())),
        preferred_element_type=jnp.float32)

    keep = gt | (eq & (cum_excl < need))
    mask_ref[:, 0, :] = keep.astype(jnp.float32)


def _mul_body(vt_ref, mask_ref, out_ref):
    patches = vt_ref[:, :, pl.ds(1, N)]
    m = mask_ref[:, 0, :]
    out_ref[...] = patches * m[:, None, :]


def kernel(vision_output, token_budget, W, b):
    vt = jnp.transpose(vision_output, (0, 2, 1))
    tb2 = token_budget.reshape(1, B)
    b2 = b.reshape(1, N)
    wt = W.T

    keep_mask3 = pl.pallas_call(
        _mask_body,
        grid=(1,),
        in_specs=[
            pl.BlockSpec((B, C, 128), lambda i: (0, 0, LAST_TILE)),
            pl.BlockSpec((1, B), lambda i: (0, 0)),
            pl.BlockSpec((C, N), lambda i: (0, 0)),
            pl.BlockSpec((1, N), lambda i: (0, 0)),
        ],
        out_specs=pl.BlockSpec((B, 1, N), lambda i: (0, 0, 0)),
        out_shape=jax.ShapeDtypeStruct((B, 1, N), jnp.float32),
    )(vt, tb2, wt, b2)

    masked_t = pl.pallas_call(
        _mul_body,
        grid=(B // B_BLK,),
        in_specs=[
            pl.BlockSpec((B_BLK, C, N + 2), lambda i: (i, 0, 0)),
            pl.BlockSpec((B_BLK, 1, N), lambda i: (i, 0, 0)),
        ],
        out_specs=pl.BlockSpec((B_BLK, C, N), lambda i: (i, 0, 0)),
        out_shape=jax.ShapeDtypeStruct((B, C, N), jnp.float32),
    )(vt, keep_mask3)

    masked = jnp.transpose(masked_t, (0, 2, 1))
    return masked, keep_mask3.reshape(B, N)


# speedup vs baseline: 4.4921x; 1.0483x over previous
import jax
import jax.numpy as jnp
from jax.experimental import pallas as pl

B, N, C = 64, 1024, 192
LAST_TILE = (N + 2) // 128
LAST_OFF = (N + 1) % 128
B_BLK = 8


def _mask_body(vtail_ref, tb_ref, wt_ref, bias_ref, mask_ref):
    br = vtail_ref[:, :, LAST_OFF]
    logits = jax.lax.dot_general(
        br, wt_ref[...],
        dimension_numbers=(((1,), (0,)), ((), ())),
        preferred_element_type=jnp.float32) + bias_ref[...]

    ii = jax.lax.broadcasted_iota(jnp.int32, (B, B), 0)
    jj = jax.lax.broadcasted_iota(jnp.int32, (B, B), 1)
    tb_col = jnp.sum(jnp.where(ii == jj, jnp.broadcast_to(tb_ref[...], (B, B)),
                               0.0), axis=1, keepdims=True)
    K = jnp.clip((tb_col * float(N)).astype(jnp.int32), 1, N)

    x = logits + 0.0
    u = jax.lax.bitcast_convert_type(x, jnp.uint32)
    key = u ^ jnp.where((u >> 31) > 0,
                        jnp.uint32(0xFFFFFFFF), jnp.uint32(0x80000000))

    t = jnp.zeros((B, 1), jnp.uint32)
    for bit in range(31, -1, -1):
        cand = t | jnp.uint32(1 << bit)
        cnt = jnp.sum((key >= cand).astype(jnp.int32), axis=1, keepdims=True)
        t = jnp.where(cnt >= K, cand, t)

    gt = key > t
    eq = key == t
    cnt_gt = jnp.sum(gt.astype(jnp.int32), axis=1, keepdims=True)
    need = (K - cnt_gt).astype(jnp.float32)

    ni = jax.lax.broadcasted_iota(jnp.int32, (N, N), 0)
    nj = jax.lax.broadcasted_iota(jnp.int32, (N, N), 1)
    ut = (ni < nj).astype(jnp.float32)
    cum_excl = jax.lax.dot_general(
        eq.astype(jnp.float32), ut,
        dimension_numbers=(((1,), (0,)), ((), ())),
        preferred_element_type=jnp.float32)

    keep = gt | (eq & (cum_excl < need))
    mask_ref[:, 0, :] = keep.astype(jnp.float32)


def _mul_body(vt_ref, mask_ref, out_ref):
    patches = vt_ref[:, :, pl.ds(1, N)]
    m = mask_ref[:, 0, :]
    out_ref[...] = patches * m[:, None, :]


def kernel(vision_output, token_budget, W, b):
    vt = jnp.transpose(vision_output, (0, 2, 1))
    tb2 = token_budget.reshape(1, B)
    b2 = b.reshape(1, N)
    wt = W.T

    keep_mask3 = pl.pallas_call(
        _mask_body,
        grid=(1,),
        in_specs=[
            pl.BlockSpec((B, C, 128), lambda i: (0, 0, LAST_TILE)),
            pl.BlockSpec((1, B), lambda i: (0, 0)),
            pl.BlockSpec((C, N), lambda i: (0, 0)),
            pl.BlockSpec((1, N), lambda i: (0, 0)),
        ],
        out_specs=pl.BlockSpec((B, 1, N), lambda i: (0, 0, 0)),
        out_shape=jax.ShapeDtypeStruct((B, 1, N), jnp.float32),
    )(vt, tb2, wt, b2)

    masked_t = pl.pallas_call(
        _mul_body,
        grid=(B // B_BLK,),
        in_specs=[
            pl.BlockSpec((B_BLK, C, N + 2), lambda i: (i, 0, 0)),
            pl.BlockSpec((B_BLK, 1, N), lambda i: (i, 0, 0)),
        ],
        out_specs=pl.BlockSpec((B_BLK, C, N), lambda i: (i, 0, 0)),
        out_shape=jax.ShapeDtypeStruct((B, C, N), jnp.float32),
    )(vt, keep_mask3)

    masked = jnp.transpose(masked_t, (0, 2, 1))
    return masked, keep_mask3.reshape(B, N)


# speedup vs baseline: 4.6463x; 1.0343x over previous
import jax
import jax.numpy as jnp
from jax.experimental import pallas as pl

B, N, C = 64, 1024, 192
LAST_TILE = (N + 2) // 128
LAST_OFF = (N + 1) % 128
B_BLK = 8


def _mask_body(vtail_ref, tb_ref, wt_ref, bias_ref, mask_ref):
    br = vtail_ref[:, :, LAST_OFF]
    logits = jax.lax.dot_general(
        br, wt_ref[...],
        dimension_numbers=(((1,), (0,)), ((), ())),
        preferred_element_type=jnp.float32) + bias_ref[...]

    ii = jax.lax.broadcasted_iota(jnp.int32, (B, B), 0)
    jj = jax.lax.broadcasted_iota(jnp.int32, (B, B), 1)
    tb_col = jnp.sum(jnp.where(ii == jj, jnp.broadcast_to(tb_ref[...], (B, B)),
                               0.0), axis=1, keepdims=True)
    K = jnp.clip((tb_col * float(N)).astype(jnp.int32), 1, N)

    x = logits + 0.0
    u = jax.lax.bitcast_convert_type(x, jnp.uint32)
    key = u ^ jnp.where((u >> 31) > 0,
                        jnp.uint32(0xFFFFFFFF), jnp.uint32(0x80000000))

    t = jnp.zeros((B, 1), jnp.uint32)
    for bit in range(31, -1, -1):
        cand = t | jnp.uint32(1 << bit)
        cnt = jnp.sum((key >= cand).astype(jnp.int32), axis=1, keepdims=True)
        t = jnp.where(cnt >= K, cand, t)

    gt = key > t
    eq = key == t
    cnt_gt = jnp.sum(gt.astype(jnp.int32), axis=1, keepdims=True)
    need = (K - cnt_gt).astype(jnp.float32)

    ni = jax.lax.broadcasted_iota(jnp.int32, (N, N), 0)
    nj = jax.lax.broadcasted_iota(jnp.int32, (N, N), 1)
    ut = (ni < nj).astype(jnp.float32)
    cum_excl = jax.lax.dot_general(
        eq.astype(jnp.float32), ut,
        dimension_numbers=(((1,), (0,)), ((), ())),
        preferred_element_type=jnp.float32)

    keep = gt | (eq & (cum_excl < need))
    mask_ref[...] = keep.astype(jnp.float32)


def _mul_body(vt_ref, mask_ref, out_ref):
    patches = vt_ref[:, :, pl.ds(1, N)]
    m = mask_ref[...]
    out_ref[...] = patches * m[:, None, :]


def kernel(vision_output, token_budget, W, b):
    vt = jnp.transpose(vision_output, (0, 2, 1))
    tb2 = token_budget.reshape(1, B)
    b2 = b.reshape(1, N)
    wt = W.T

    keep_mask = pl.pallas_call(
        _mask_body,
        grid=(1,),
        in_specs=[
            pl.BlockSpec((B, C, 128), lambda i: (0, 0, LAST_TILE)),
            pl.BlockSpec((1, B), lambda i: (0, 0)),
            pl.BlockSpec((C, N), lambda i: (0, 0)),
            pl.BlockSpec((1, N), lambda i: (0, 0)),
        ],
        out_specs=pl.BlockSpec((B, N), lambda i: (0, 0)),
        out_shape=jax.ShapeDtypeStruct((B, N), jnp.float32),
    )(vt, tb2, wt, b2)

    masked_t = pl.pallas_call(
        _mul_body,
        grid=(B // B_BLK,),
        in_specs=[
            pl.BlockSpec((B_BLK, C, N + 2), lambda i: (i, 0, 0)),
            pl.BlockSpec((B_BLK, N), lambda i: (i, 0)),
        ],
        out_specs=pl.BlockSpec((B_BLK, C, N), lambda i: (i, 0, 0)),
        out_shape=jax.ShapeDtypeStruct((B, C, N), jnp.float32),
    )(vt, keep_mask)

    masked = jnp.transpose(masked_t, (0, 2, 1))
    return masked, keep_mask
